# Initial kernel scaffold; baseline (speedup 1.0000x reference)
import jax
import jax.numpy as jnp
from jax.experimental import pallas as pl


def kernel(x):
    raise NotImplementedError("write your pallas kernel here")


# speedup vs baseline: 11.9551x; 11.9551x over previous
import functools

import jax
import jax.numpy as jnp
from jax import lax
from jax.experimental import pallas as pl
from jax.experimental.pallas import tpu as pltpu
from jax.experimental.pallas import tpu_sc as plsc

N = 2
HW = 4097
D = 768
K = 1024
L = 16
NC = 2
NS = 16
NW = NC * NS
CH_PER_W = (N * D) // NW
NSLAB = CH_PER_W // L
NB = 256


def _topk_body(x_hbm, out_hbm, buf, hist):
    wid = lax.axis_index("s") * NC + lax.axis_index("c")
    n = wid // (D // CH_PER_W)
    dbase = (wid % (D // CH_PER_W)) * CH_PER_W

    lane = lax.iota(jnp.int32, L)
    ones = jnp.ones((L,), jnp.int32)
    zeros = jnp.zeros((L,), jnp.int32)
    kfull = jnp.full((L,), K, jnp.int32)

    def zero_body(b, _):
        hist[b] = zeros
        return 0

    lax.fori_loop(0, NB, zero_body, 0)

    for s in range(NSLAB):
        d = dbase + s * L
        pltpu.sync_copy(x_hbm.at[n, :, pl.ds(d, L)], buf)

        kk = kfull
        t_prefix = zeros
        for p in range(3):
            shift = 23 - 8 * p

            if p == 0:
                def hist_body(i, _):
                    bits = plsc.bitcast(buf[i], jnp.int32) & 0x7FFFFFFF
                    b = bits >> shift
                    plsc.addupdate_scatter(hist, [b, lane], ones)
                    return 0
            else:
                def hist_body(i, _, shift=shift, t_prefix=t_prefix):
                    bits = plsc.bitcast(buf[i], jnp.int32) & 0x7FFFFFFF
                    b = (bits >> shift) & 0xFF
                    m = (bits >> (shift + 8)) == t_prefix
                    plsc.addupdate_scatter(hist, [b, lane], ones, mask=m)
                    return 0

            lax.fori_loop(1, HW, hist_body, 0)

            def scan_body(j, carry):
                acc, bsel, above = carry
                b = NB - 1 - j
                h = hist[b]
                hist[b] = zeros
                acc2 = acc + h
                crossed = (acc < kk) & (acc2 >= kk)
                bsel = jnp.where(crossed, zeros + b, bsel)
                above = jnp.where(crossed, acc, above)
                return (acc2, bsel, above)

            _, bsel, above = lax.fori_loop(0, NB, scan_body,
                                           (zeros, zeros, zeros))
            kk = kk - above
            t_prefix = (t_prefix << 8) | bsel

        t24 = t_prefix

        def mask_body(i, _):
            v = buf[i]
            bits = plsc.bitcast(v, jnp.int32) & 0x7FFFFFFF
            keep = (bits >> 7) >= t24
            buf[i] = jnp.where(keep, v, jnp.zeros((L,), jnp.float32))
            return 0

        lax.fori_loop(1, HW, mask_body, 0)

        pltpu.sync_copy(buf, out_hbm.at[n, :, pl.ds(d, L)])


_topk_call = functools.partial(
    pl.kernel,
    out_type=jax.ShapeDtypeStruct((N, HW, D), jnp.float32),
    mesh=plsc.VectorSubcoreMesh(core_axis_name="c", subcore_axis_name="s"),
    scratch_types=[
        pltpu.VMEM((HW, L), jnp.float32),
        pltpu.VMEM((NB, L), jnp.int32),
    ],
    compiler_params=pltpu.CompilerParams(use_tc_tiling_on_sc=False,
                                         needs_layout_passes=False),
)(_topk_body)


@jax.jit
def kernel(x):
    return _topk_call(x)


# speedup vs baseline: 18.5901x; 1.5550x over previous
import functools

import jax
import jax.numpy as jnp
from jax import lax
from jax.experimental import pallas as pl
from jax.experimental.pallas import tpu as pltpu
from jax.experimental.pallas import tpu_sc as plsc

N = 2
HW = 4097
D = 768
K = 1024
L = 16
NC = 2
NS = 16
NW = NC * NS
CH_PER_W = (N * D) // NW
NSLAB = CH_PER_W // L
NB = 256


def _topk_body(x_hbm, out_hbm, buf, hist):
    wid = lax.axis_index("s") * NC + lax.axis_index("c")
    n = wid // (D // CH_PER_W)
    dbase = (wid % (D // CH_PER_W)) * CH_PER_W

    lane = lax.iota(jnp.int32, L)
    ones = jnp.ones((L,), jnp.int32)
    zeros = jnp.zeros((L,), jnp.int32)
    kfull = jnp.full((L,), K, jnp.int32)

    @plsc.parallel_loop(0, NB, unroll=8)
    def _(b):
        hist[b] = zeros

    for s in range(NSLAB):
        d = dbase + s * L
        pltpu.sync_copy(x_hbm.at[n, :, pl.ds(d, L)], buf)

        kk = kfull
        t_prefix = zeros
        for p in range(3):
            shift = 23 - 8 * p

            if p == 0:
                @plsc.parallel_loop(1, HW, unroll=8)
                def _(i):
                    bits = plsc.bitcast(buf[i], jnp.int32) & 0x7FFFFFFF
                    b = bits >> shift
                    plsc.addupdate_scatter(hist, [b, lane], ones)
            else:
                @plsc.parallel_loop(1, HW, unroll=8)
                def _(i, shift=shift, t_prefix=t_prefix):
                    bits = plsc.bitcast(buf[i], jnp.int32) & 0x7FFFFFFF
                    b = (bits >> shift) & 0xFF
                    m = (bits >> (shift + 8)) == t_prefix
                    plsc.addupdate_scatter(hist, [b, lane], ones, mask=m)

            @plsc.parallel_loop(0, NB, unroll=4, carry=(zeros, zeros, zeros))
            def scan_out(j, carry):
                acc, bsel, above = carry
                b = NB - 1 - j
                h = hist[b]
                hist[b] = zeros
                acc2 = acc + h
                crossed = (acc < kk) & (acc2 >= kk)
                bsel = jnp.where(crossed, zeros + b, bsel)
                above = jnp.where(crossed, acc, above)
                return (acc2, bsel, above)

            _, bsel, above = scan_out
            kk = kk - above
            t_prefix = (t_prefix << 8) | bsel

        t24 = t_prefix

        @plsc.parallel_loop(1, HW, unroll=8)
        def _(i, t24=t24):
            v = buf[i]
            bits = plsc.bitcast(v, jnp.int32) & 0x7FFFFFFF
            keep = (bits >> 7) >= t24
            buf[i] = jnp.where(keep, v, jnp.zeros((L,), jnp.float32))

        pltpu.sync_copy(buf, out_hbm.at[n, :, pl.ds(d, L)])


_topk_call = functools.partial(
    pl.kernel,
    out_type=jax.ShapeDtypeStruct((N, HW, D), jnp.float32),
    mesh=plsc.VectorSubcoreMesh(core_axis_name="c", subcore_axis_name="s"),
    scratch_types=[
        pltpu.VMEM((HW, L), jnp.float32),
        pltpu.VMEM((NB, L), jnp.int32),
    ],
    compiler_params=pltpu.CompilerParams(use_tc_tiling_on_sc=False,
                                         needs_layout_passes=False),
)(_topk_body)


@jax.jit
def kernel(x):
    return _topk_call(x)


# speedup vs baseline: 44.7709x; 2.4083x over previous
import functools

import jax
import jax.numpy as jnp
from jax import lax
from jax.experimental import pallas as pl
from jax.experimental.pallas import tpu as pltpu
from jax.experimental.pallas import tpu_sc as plsc

N = 2
HW = 4097
D = 768
K = 1024
L = 16
NC = 2
NS = 16
CH_PER_W = (N * D) // (NC * NS)
NSLAB = CH_PER_W // L
NB = 256
HALF = 2048


def _topk_body(xt_hbm, out_hbm, cbuf, tslab, hist, sliver):
    wid = lax.axis_index("s") * NC + lax.axis_index("c")
    n = wid // (D // CH_PER_W)
    cbase = (wid % (D // CH_PER_W)) * CH_PER_W

    lane = lax.iota(jnp.int32, L)
    lane16 = lane * 16
    ones = jnp.ones((L,), jnp.int32)
    zeros = jnp.zeros((L,), jnp.int32)
    zf = jnp.zeros((L,), jnp.float32)
    kfull = jnp.full((L,), K, jnp.int32)

    @plsc.parallel_loop(0, NB, unroll=8)
    def _(b):
        hist[pl.ds(b * L, L)] = zeros

    for s in range(NSLAB):
        c0 = cbase + s * L

        for h in range(2):
            pltpu.sync_copy(xt_hbm.at[n, pl.ds(c0, L), pl.ds(h * HALF, HALF)],
                            cbuf)
            for c in range(L):
                base_const = h * HALF * L + c

                @plsc.parallel_loop(0, HALF // L, unroll=8)
                def _(j, c=c, base_const=base_const):
                    v = cbuf[c, pl.ds(j * L, L)]
                    idx = lane16 + (base_const + j * L * L)
                    plsc.store_scatter(tslab, [idx], v)

        pltpu.sync_copy(xt_hbm.at[n, pl.ds(c0, L), pl.ds(HW - 1, 1)], sliver)
        vlast = plsc.load_gather(sliver, [lane, zeros])
        plsc.store_scatter(tslab, [lane + (HW - 1) * L], vlast)

        kk = kfull
        t_prefix = zeros
        for p in range(3):
            shift = 23 - 8 * p

            if p == 0:
                @plsc.parallel_loop(1, HW, unroll=8)
                def _(i):
                    bits = plsc.bitcast(tslab[pl.ds(i * L, L)],
                                        jnp.int32) & 0x7FFFFFFF
                    b = bits >> shift
                    plsc.addupdate_scatter(hist, [b * L + lane], ones)
            else:
                @plsc.parallel_loop(1, HW, unroll=8)
                def _(i, shift=shift, t_prefix=t_prefix):
                    bits = plsc.bitcast(tslab[pl.ds(i * L, L)],
                                        jnp.int32) & 0x7FFFFFFF
                    b = (bits >> shift) & 0xFF
                    m = (bits >> (shift + 8)) == t_prefix
                    plsc.addupdate_scatter(hist, [b * L + lane], ones, mask=m)

            @plsc.parallel_loop(0, NB, unroll=4, carry=(zeros, zeros, zeros))
            def scan_out(j, carry):
                acc, bsel, above = carry
                b = NB - 1 - j
                h = hist[pl.ds(b * L, L)]
                hist[pl.ds(b * L, L)] = zeros
                acc2 = acc + h
                crossed = (acc < kk) & (acc2 >= kk)
                bsel = jnp.where(crossed, zeros + b, bsel)
                above = jnp.where(crossed, acc, above)
                return (acc2, bsel, above)

            _, bsel, above = scan_out
            kk = kk - above
            t_prefix = (t_prefix << 8) | bsel

        for h in range(2):
            for c in range(L):
                tvec = jnp.full((L,), 0, jnp.int32) + t_prefix[c]
                cvec = zeros + c
                base_const = h * HALF * L + c

                @plsc.parallel_loop(0, HALF // L, unroll=8)
                def _(j, c=c, base_const=base_const, tvec=tvec, cvec=cvec):
                    idx = lane16 + (base_const + j * L * L)
                    v = plsc.load_gather(tslab, [idx])
                    bits = plsc.bitcast(v, jnp.int32) & 0x7FFFFFFF
                    keep = ((bits >> 7) >= tvec) | (idx == cvec)
                    cbuf[c, pl.ds(j * L, L)] = jnp.where(keep, v, zf)

            pltpu.sync_copy(cbuf,
                            out_hbm.at[n, pl.ds(c0, L), pl.ds(h * HALF, HALF)])

        vlast = plsc.load_gather(tslab, [lane + (HW - 1) * L])
        lbits = plsc.bitcast(vlast, jnp.int32) & 0x7FFFFFFF
        vmasked = jnp.where((lbits >> 7) >= t_prefix, vlast, zf)
        plsc.store_scatter(sliver, [lane, zeros], vmasked)
        pltpu.sync_copy(sliver, out_hbm.at[n, pl.ds(c0, L), pl.ds(HW - 1, 1)])


_topk_call = functools.partial(
    pl.kernel,
    out_type=jax.ShapeDtypeStruct((N, D, HW), jnp.float32),
    mesh=plsc.VectorSubcoreMesh(core_axis_name="c", subcore_axis_name="s"),
    scratch_types=[
        pltpu.VMEM((L, HALF), jnp.float32),
        pltpu.VMEM((HW * L,), jnp.float32),
        pltpu.VMEM((NB * L,), jnp.int32),
        pltpu.VMEM((L, 1), jnp.float32),
    ],
    compiler_params=pltpu.CompilerParams(needs_layout_passes=False),
)(_topk_body)


@jax.jit
def kernel(x):
    xt = jnp.transpose(x, (0, 2, 1))
    yt = _topk_call(xt)
    return jnp.transpose(yt, (0, 2, 1))
